# baseline (speedup 1.0000x reference)
import functools

import jax
import jax.numpy as jnp
from jax import lax
from jax.experimental import pallas as pl
from jax.experimental.pallas import tpu as pltpu
from jax.experimental.pallas import tpu_sc as plsc

_DP = 128
_B_TOTAL = 4096 * 200
_NW = 32
_B_PER_W = _B_TOTAL // _NW
_CHUNK = 640
_N_CHUNKS = _B_PER_W // _CHUNK
_N_PAIRS = _N_CHUNKS // 2

_mesh = plsc.VectorSubcoreMesh(core_axis_name="c", subcore_axis_name="s")


@functools.partial(
    pl.kernel,
    out_type=jax.ShapeDtypeStruct((_B_TOTAL, _DP), jnp.float32),
    mesh=_mesh,
    scratch_types=[
        pltpu.VMEM((_B_PER_W,), jnp.int32),
        pltpu.VMEM((_CHUNK, 64), jnp.float32),
        pltpu.VMEM((_CHUNK, 64), jnp.float32),
        pltpu.SemaphoreType.DMA,
        pltpu.SemaphoreType.DMA,
        pltpu.SemaphoreType.DMA,
        pltpu.SemaphoreType.DMA,
    ],
    compiler_params=pltpu.CompilerParams(use_tc_tiling_on_sc=False),
)
def _gather_kernel(table_hbm, idx_hbm, out_hbm, idx_v, rows0, rows1,
                   g0, g1, s0, s1):
    wid = lax.axis_index("s") * 2 + lax.axis_index("c")
    base = wid * _B_PER_W
    pltpu.sync_copy(idx_hbm.at[pl.ds(base, _B_PER_W)], idx_v)

    def g_start(i, buf, sem):
        pltpu.async_copy(table_hbm.at[idx_v.at[pl.ds(i * _CHUNK, _CHUNK)]],
                         buf, sem)

    def g_wait(buf, sem):
        pltpu.make_async_copy(table_hbm.at[idx_v.at[pl.ds(0, _CHUNK)]],
                              buf, sem).wait()

    def s_start(i, buf, sem):
        pltpu.async_copy(buf,
                         out_hbm.at[pl.ds(base + i * _CHUNK, _CHUNK),
                                    pl.ds(0, 64)],
                         sem)

    def s_wait(buf, sem):
        pltpu.make_async_copy(buf,
                              out_hbm.at[pl.ds(base, _CHUNK), pl.ds(0, 64)],
                              sem).wait()

    g_start(0, rows0, g0)
    g_start(1, rows1, g1)

    def body(p, carry):
        i0 = 2 * p
        g_wait(rows0, g0)
        s_start(i0, rows0, s0)
        g_wait(rows1, g1)
        s_start(i0 + 1, rows1, s1)

        @pl.when(p + 1 < _N_PAIRS)
        def _prefetch():
            s_wait(rows0, s0)
            g_start(i0 + 2, rows0, g0)
            s_wait(rows1, s1)
            g_start(i0 + 3, rows1, g1)

        return carry

    lax.fori_loop(0, _N_PAIRS, body, 0)
    s_wait(rows0, s0)
    s_wait(rows1, s1)


def kernel(weight, token_ids):
    wlin = jax.lax.optimization_barrier(weight.reshape(-1))
    wtab = wlin.reshape(1000000, 64)
    flat_ids = token_ids.reshape(-1).astype(jnp.int32)
    outp = _gather_kernel(wtab, flat_ids)
    out3 = outp.reshape(token_ids.shape + (_DP,))
    return out3[:, :, : weight.shape[1]]
